# baseline (speedup 1.0000x reference)
import jax
import jax.numpy as jnp
from jax import lax
from jax.experimental import pallas as pl
from jax.experimental.pallas import tpu as pltpu
from jax.experimental.pallas import tpu_sc as plsc


_BN = 1024
_CH = 128
_NS = 16
_BC = 128


def _assign_body(x_ref, m_ref, idx_ref):
    x = x_ref[...]
    m = m_ref[...]
    bn = x.shape[0]
    nc = m.shape[0]

    x2 = jnp.sum(x * x, axis=1)
    m2 = jnp.sum(m * m, axis=1)
    dot = jax.lax.dot_general(
        x, m, (((1,), (1,)), ((), ())),
        preferred_element_type=jnp.float32)
    d2 = (x2[:, None] + m2[None, :]) - 2.0 * dot
    d2 = jnp.maximum(d2, 0.0)
    mn = jnp.min(d2, axis=1, keepdims=True)
    lane = jax.lax.broadcasted_iota(jnp.int32, (bn, nc), 1)
    idx_ref[...] = jnp.min(jnp.where(d2 == mn, lane, nc), axis=1)


def _bcast(vec, r):
    return lax.gather(
        vec, jnp.full((16, 1), r, jnp.int32),
        lax.GatherDimensionNumbers(
            offset_dims=(), collapsed_slice_dims=(0,), start_index_map=(0,)),
        slice_sizes=(1,),
        mode=lax.GatherScatterMode.PROMISE_IN_BOUNDS)


def _fit_sc_body(x_hbm, z_hbm, idx_hbm, ps_hbm, cnt_hbm,
                 idx_all, x_a, x_b, acc, acc_cnt,
                 sem_z, sem_a, sem_b):
    c = lax.axis_index("c")
    s = lax.axis_index("s")
    n, k = x_hbm.shape
    nc = acc_cnt.shape[0]
    kt = k // 2
    rpt = n // _NS
    rbase = s * rpt
    cbase = c * kt
    nch = rpt // _CH

    hz = pltpu.async_copy(z_hbm.at[c], acc, sem_z)
    hcz = pltpu.async_copy(z_hbm.at[c, pl.ds(0, nc)], acc_cnt, sem_z)
    hi = pltpu.async_copy(idx_hbm.at[pl.ds(rbase, rpt)], idx_all, sem_z)

    bufs = (x_a, x_b)
    sems = (sem_a, sem_b)
    hx = [None, None]
    hx[0] = pltpu.async_copy(
        x_hbm.at[pl.ds(rbase, _CH), pl.ds(cbase, kt)], x_a, sem_a)
    hz.wait()
    hcz.wait()
    hi.wait()

    colv = [j0 + lax.iota(jnp.int32, 16) for j0 in range(0, kt, 16)]
    nblk = kt // 16
    ones = jnp.full((16,), 1.0, jnp.float32)

    for t in range(nch):
        b = t & 1
        if t + 1 < nch:
            hx[1 - b] = pltpu.async_copy(
                x_hbm.at[pl.ds(rbase + (t + 1) * _CH, _CH), pl.ds(cbase, kt)],
                bufs[1 - b], sems[1 - b])
        hx[b].wait()
        xs = bufs[b]

        def group(g, _):
            grp = idx_all[pl.ds(t * _CH + g * 16, 16)]
            plsc.addupdate_scatter(acc_cnt, [grp], ones)
            gb = grp * kt
            for r in range(16):
                bc = _bcast(gb, r)
                row = g * 16 + r
                for i in range(nblk):
                    vals = xs[row, pl.ds(i * 16, 16)]
                    plsc.addupdate_scatter(acc, [bc + colv[i]], vals)
            return 0

        lax.fori_loop(0, _CH // 16, group, 0)

    hp = pltpu.async_copy(acc, ps_hbm.at[c * _NS + s], sem_a)
    hc = pltpu.async_copy(acc_cnt, cnt_hbm.at[c * _NS + s], sem_b)
    hp.wait()
    hc.wait()


def _update_body(ps_ref, cnt_ref, m_ref, out_ref):
    ps = ps_ref[...]
    cnt = jnp.sum(cnt_ref[...], axis=1, keepdims=True) * 0.5
    lo = jnp.sum(ps[0], axis=0)
    hi = jnp.sum(ps[1], axis=0)
    sums = jnp.concatenate([lo, hi], axis=1)
    denom = jnp.maximum(cnt, 1.0)
    out_ref[...] = jnp.where(cnt == 0.0, m_ref[...], sums / denom)


@jax.jit
def _one_iter(Xr, Z, M):
    n, k = Xr.shape
    nc = M.shape[0]
    kt = k // 2
    nb = n // _BN
    idx = pl.pallas_call(
        _assign_body,
        grid=(nb,),
        in_specs=[
            pl.BlockSpec((_BN, k), lambda i: (i, 0)),
            pl.BlockSpec((nc, k), lambda i: (0, 0)),
        ],
        out_specs=pl.BlockSpec((_BN,), lambda i: (i,)),
        out_shape=jax.ShapeDtypeStruct((n,), jnp.int32),
    )(Xr, M)

    fit = pl.kernel(
        _fit_sc_body,
        out_type=[
            jax.ShapeDtypeStruct((2 * _NS, nc * kt), jnp.float32),
            jax.ShapeDtypeStruct((2 * _NS, nc), jnp.float32),
        ],
        mesh=plsc.VectorSubcoreMesh(core_axis_name="c", subcore_axis_name="s"),
        scratch_types=[
            pltpu.VMEM((n // _NS,), jnp.int32),
            pltpu.VMEM((_CH, kt), jnp.float32),
            pltpu.VMEM((_CH, kt), jnp.float32),
            pltpu.VMEM((nc * kt,), jnp.float32),
            pltpu.VMEM((nc,), jnp.float32),
            pltpu.SemaphoreType.DMA,
            pltpu.SemaphoreType.DMA,
            pltpu.SemaphoreType.DMA,
        ],
        compiler_params=pltpu.CompilerParams(needs_layout_passes=False),
    )
    ps, cnt = fit(Xr, Z, idx)
    ps = ps.reshape(2, _NS, nc, kt)
    cnt = cnt.T
    return pl.pallas_call(
        _update_body,
        grid=(nc // _BC,),
        in_specs=[
            pl.BlockSpec((2, _NS, _BC, kt), lambda i: (0, 0, i, 0)),
            pl.BlockSpec((_BC, 2 * _NS), lambda i: (i, 0)),
            pl.BlockSpec((_BC, k), lambda i: (i, 0)),
        ],
        out_specs=pl.BlockSpec((_BC, k), lambda i: (i, 0)),
        out_shape=jax.ShapeDtypeStruct((nc, k), jnp.float32),
    )(ps, cnt, M)


def kernel(X, mu, niter):
    nc, _, k = mu.shape
    Xr = X.reshape(-1, k)
    M0 = mu[:, 0, :]
    Z = jnp.zeros((2, nc * (k // 2)), jnp.float32)
    Mf = jax.lax.fori_loop(0, niter, lambda t, M: _one_iter(Xr, Z, M), M0)
    return Mf[:, None, :]
